# Initial kernel scaffold; baseline (speedup 1.0000x reference)
import jax
import jax.numpy as jnp
from jax.experimental import pallas as pl


def kernel(input, weight_int8, weight_scale):
    raise NotImplementedError("write your pallas kernel here")


# speedup vs baseline: 5.6389x; 5.6389x over previous
import jax
import jax.numpy as jnp
from jax import lax
from jax.experimental import pallas as pl
from jax.experimental.pallas import tpu as pltpu
from jax.experimental.pallas import tpu_sc as plsc

NUM_EMBEDDINGS = 1000000
EMBEDDING_DIM = 32
BATCH = 16384
N_FIELDS = 26

NC, NS, L = 2, 16, 16
NW = NC * NS
TOTAL = BATCH * N_FIELDS
PER_W = TOTAL // NW
CHUNK = 1024
N_CHUNKS = PER_W // CHUNK
WORDS = EMBEDDING_DIM // 4


def _body(idx_hbm, w32_hbm, scale_hbm, out_hbm,
          idx_v, rows_v, scales_v, out_v, sem_r, sem_s):
    wid = lax.axis_index("s") * NC + lax.axis_index("c")
    base = wid * PER_W

    lane = lax.iota(jnp.int32, L)
    halfsel = lane >> 3
    outperm = [halfsel * 32 + (lane & 7) * 4 + b for b in range(4)]
    colperm = lane & 7

    for c in range(N_CHUNKS):
        cbase = base + c * CHUNK
        pltpu.sync_copy(idx_hbm.at[pl.ds(cbase, CHUNK)], idx_v)
        cp_r = pltpu.async_copy(w32_hbm.at[idx_v], rows_v, sem_r)
        cp_s = pltpu.async_copy(scale_hbm.at[idx_v], scales_v, sem_s)
        cp_r.wait()
        cp_s.wait()

        def dequant(j, carry):
            row_idx = halfsel + 2 * j
            w = plsc.load_gather(rows_v, [row_idx, colperm])
            s = plsc.load_gather(scales_v, [row_idx])
            obase = j * 64
            for b in range(4):
                if b == 3:
                    v = lax.shift_right_arithmetic(w, 24)
                else:
                    v = lax.shift_right_arithmetic(
                        lax.shift_left(w, (3 - b) * 8), 24)
                plsc.store_scatter(out_v, [outperm[b] + obase],
                                   v.astype(jnp.float32) * s)
            return carry

        lax.fori_loop(0, CHUNK // 2, dequant, 0)
        pltpu.sync_copy(out_v, out_hbm.at[pl.ds(cbase * EMBEDDING_DIM,
                                                CHUNK * EMBEDDING_DIM)])


@jax.jit
def _run(idx_flat, w32, weight_scale):
    mesh = plsc.VectorSubcoreMesh(core_axis_name="c", subcore_axis_name="s")
    return pl.kernel(
        _body,
        out_type=jax.ShapeDtypeStruct((TOTAL * EMBEDDING_DIM,), jnp.float32),
        mesh=mesh,
        compiler_params=pltpu.CompilerParams(needs_layout_passes=False,
                                             use_tc_tiling_on_sc=False),
        scratch_types=[
            pltpu.VMEM((CHUNK,), jnp.int32),
            pltpu.VMEM((CHUNK, WORDS), jnp.int32),
            pltpu.VMEM((CHUNK,), jnp.float32),
            pltpu.VMEM((CHUNK * EMBEDDING_DIM,), jnp.float32),
            pltpu.SemaphoreType.DMA,
            pltpu.SemaphoreType.DMA,
        ],
    )(idx_flat, w32, weight_scale)


def kernel(input, weight_int8, weight_scale):
    idx_flat = input.reshape(-1).astype(jnp.int32)
    w32 = lax.bitcast_convert_type(
        weight_int8.reshape(NUM_EMBEDDINGS, WORDS, 4), jnp.int32)
    out = _run(idx_flat, w32, weight_scale)
    return out.reshape(BATCH, N_FIELDS, EMBEDDING_DIM)


# speedup vs baseline: 6.9114x; 1.2257x over previous
import jax
import jax.numpy as jnp
from jax import lax
from jax.experimental import pallas as pl
from jax.experimental.pallas import tpu as pltpu
from jax.experimental.pallas import tpu_sc as plsc

NUM_EMBEDDINGS = 1000000
EMBEDDING_DIM = 32
BATCH = 16384
N_FIELDS = 26

NC, NS, L = 2, 16, 16
NW = NC * NS
TOTAL = BATCH * N_FIELDS
PER_W = TOTAL // NW
CHUNK = 1024
N_CHUNKS = PER_W // CHUNK
WORDS = EMBEDDING_DIM // 4


def _body(idx_hbm, w8_hbm, scale_hbm, out_hbm,
          idx_v0, idx_v1, rows_v0, rows_v1, scales_v0, scales_v1,
          out_v0, out_v1,
          sem_r0, sem_r1, sem_s0, sem_s1, sem_o0, sem_o1):
    idx_b = (idx_v0, idx_v1)
    rows_b = (rows_v0, rows_v1)
    scales_b = (scales_v0, scales_v1)
    out_b = (out_v0, out_v1)
    sem_r = (sem_r0, sem_r1)
    sem_s = (sem_s0, sem_s1)
    sem_o = (sem_o0, sem_o1)

    wid = lax.axis_index("s") * NC + lax.axis_index("c")
    base = wid * PER_W

    lane = lax.iota(jnp.int32, L)
    cols = [jnp.full((L,), g, jnp.int32) for g in range(WORDS)]

    def gathers(c, bi):
        cbase = base + c * CHUNK
        f_c = cbase // BATCH
        b0 = cbase % BATCH
        pltpu.sync_copy(idx_hbm.at[f_c, pl.ds(b0, CHUNK)], idx_b[bi])
        return (pltpu.async_copy(w8_hbm.at[idx_b[bi]], rows_b[bi],
                                 sem_r[bi]),
                pltpu.async_copy(scale_hbm.at[idx_b[bi]], scales_b[bi],
                                 sem_s[bi]))

    def dequant(rows, scales, out):
        @plsc.parallel_loop(0, CHUNK // L, unroll=2)
        def _deq(j):
            lk = L * j + lane
            s = scales[pl.ds(L * j, L)]
            for g in range(WORDS):
                w = plsc.load_gather(rows, [lk, cols[g]])
                for b in range(4):
                    if b == 3:
                        v = lax.shift_right_arithmetic(w, 24)
                    else:
                        v = lax.shift_right_arithmetic(
                            lax.shift_left(w, (3 - b) * 8), 24)
                    out[4 * g + b, pl.ds(L * j, L)] = \
                        v.astype(jnp.float32) * s

    cps = [None, None]
    ocs = [None, None]
    cps[0] = gathers(0, 0)
    for c in range(N_CHUNKS):
        bi = c % 2
        if c + 1 < N_CHUNKS:
            cps[1 - bi] = gathers(c + 1, 1 - bi)
        cps[bi][0].wait()
        cps[bi][1].wait()
        if ocs[bi] is not None:
            ocs[bi].wait()
        dequant(rows_b[bi], scales_b[bi], out_b[bi])
        cbase = base + c * CHUNK
        f_c = cbase // BATCH
        b0 = cbase % BATCH
        ocs[bi] = pltpu.async_copy(
            out_b[bi],
            out_hbm.at[pl.ds(f_c * EMBEDDING_DIM, EMBEDDING_DIM),
                       pl.ds(b0, CHUNK)],
            sem_o[bi])
    ocs[0].wait()
    ocs[1].wait()


@jax.jit
def _run(idx_t, w8, weight_scale):
    mesh = plsc.VectorSubcoreMesh(core_axis_name="c", subcore_axis_name="s")
    return pl.kernel(
        _body,
        out_type=jax.ShapeDtypeStruct((N_FIELDS * EMBEDDING_DIM, BATCH),
                                      jnp.float32),
        mesh=mesh,
        compiler_params=pltpu.CompilerParams(needs_layout_passes=False,
                                             use_tc_tiling_on_sc=False),
        scratch_types=[
            pltpu.VMEM((CHUNK,), jnp.int32),
            pltpu.VMEM((CHUNK,), jnp.int32),
            pltpu.VMEM((CHUNK, WORDS), jnp.int32),
            pltpu.VMEM((CHUNK, WORDS), jnp.int32),
            pltpu.VMEM((CHUNK,), jnp.float32),
            pltpu.VMEM((CHUNK,), jnp.float32),
            pltpu.VMEM((EMBEDDING_DIM, CHUNK), jnp.float32),
            pltpu.VMEM((EMBEDDING_DIM, CHUNK), jnp.float32),
            pltpu.SemaphoreType.DMA,
            pltpu.SemaphoreType.DMA,
            pltpu.SemaphoreType.DMA,
            pltpu.SemaphoreType.DMA,
            pltpu.SemaphoreType.DMA,
            pltpu.SemaphoreType.DMA,
        ],
    )(idx_t, w8, weight_scale)


def kernel(input, weight_int8, weight_scale):
    idx_t = input.T
    w32 = lax.bitcast_convert_type(
        weight_int8.reshape(NUM_EMBEDDINGS, WORDS, 4), jnp.int32)
    out = _run(idx_t, w32, weight_scale)
    return out.reshape(N_FIELDS, EMBEDDING_DIM, BATCH).transpose(2, 0, 1)
